# baseline (device time: 47450 ns/iter reference)
import jax
import jax.numpy as jnp
from jax import lax
from jax.experimental import pallas as pl
from jax.experimental.pallas import tpu as pltpu

N_DEV = 8
FWD = 3
BWD = 3
B_LOC = 2
H_LOC = 4
SQ = 128
SKV = 128
HQ = 32
DH = 64
D_MODEL = 512
D_CHUNK = H_LOC * DH


def kernel(x, Wq, K_ext, V_ext, Wo):
    my = lax.axis_index("i")
    k2 = K_ext.reshape(K_ext.shape[0], SKV, HQ * DH)
    v2 = V_ext.reshape(V_ext.shape[0], SKV, HQ * DH)
    k_b = lax.dynamic_slice_in_dim(k2, my * B_LOC, B_LOC, axis=0).astype(
        jnp.bfloat16)
    v_b = lax.dynamic_slice_in_dim(v2, my * B_LOC, B_LOC, axis=0).astype(
        jnp.bfloat16)
    x_b = x.astype(jnp.bfloat16)
    chunk = jnp.stack(
        [Wq.astype(jnp.bfloat16), Wo.T.astype(jnp.bfloat16)])

    def body(x_ref, chunk_ref, k_hbm, v_hbm, out_ref,
             comm, kvbuf, send_f, recv_sems, send_b, send_z, kv_sems):
        my_pos = lax.axis_index("i")
        left = jnp.mod(my_pos - 1, N_DEV)
        right = jnp.mod(my_pos + 1, N_DEV)
        partner = jnp.mod(my_pos + 4, N_DEV)

        kv_dmas = []
        for t, hbm in ((0, k_hbm), (1, v_hbm)):
            dma = pltpu.make_async_copy(hbm, kvbuf.at[t], kv_sems.at[t])
            dma.start()
            kv_dmas.append(dma)

        barrier_sem = pltpu.get_barrier_semaphore()
        for nbr in (left, right, partner):
            pl.semaphore_signal(
                barrier_sem, inc=1,
                device_id=(nbr,), device_id_type=pl.DeviceIdType.MESH,
            )
        pl.semaphore_wait(barrier_sem, 3)

        comm[0] = chunk_ref[...]

        qb = lax.broadcasted_iota(jnp.int32, (SQ, SQ), 0) // 64
        kb = lax.broadcasted_iota(jnp.int32, (SQ, SQ), 1) // 64
        mask = (qb == kb) | ((kb % 4) == (qb % 4))

        sends = []

        def start_send(src_slot, dst_slot, half, sem, target):
            r = pltpu.make_async_remote_copy(
                src_ref=comm.at[src_slot, half],
                dst_ref=comm.at[dst_slot, half],
                send_sem=sem, recv_sem=recv_sems.at[dst_slot, half],
                device_id=(target,), device_id_type=pl.DeviceIdType.MESH,
            )
            r.start()
            sends.append(r)

        def wait_recv(slot, half):
            pltpu.make_async_remote_copy(
                src_ref=comm.at[0, half], dst_ref=comm.at[slot, half],
                send_sem=send_f.at[0, half],
                recv_sem=recv_sems.at[slot, half],
                device_id=(my_pos,), device_id_type=pl.DeviceIdType.MESH,
            ).wait_recv()

        def compute_chunk(slot, origin, first=False):
            src = jnp.mod(origin, N_DEV)
            lane0 = src * D_CHUNK
            wq_c = comm[slot, 0]
            woT_c = comm[slot, 1]
            for b in range(B_LOC):
                q_full = (jnp.dot(x_ref[b], wq_c,
                                  preferred_element_type=jnp.float32)
                          ).astype(jnp.bfloat16)
                k4 = kvbuf[0, b, :, pl.ds(lane0, D_CHUNK)]
                v4 = kvbuf[1, b, :, pl.ds(lane0, D_CHUNK)]
                ctx_parts = []
                for h in range(H_LOC):
                    sl = slice(h * DH, (h + 1) * DH)
                    sc = lax.dot_general(
                        q_full[:, sl], k4[:, sl], (((1,), (1,)), ((), ())),
                        preferred_element_type=jnp.float32) * 0.125
                    sc = jnp.where(mask, sc, -1e9)
                    m = jnp.max(sc, axis=-1, keepdims=True)
                    w = jnp.exp(sc - m)
                    w = (w / jnp.sum(w, axis=-1, keepdims=True)
                         ).astype(jnp.bfloat16)
                    ctx_parts.append(
                        jnp.dot(w, v4[:, sl],
                                preferred_element_type=jnp.float32))
                ctx = jnp.concatenate(ctx_parts, axis=-1).astype(jnp.bfloat16)
                contrib = lax.dot_general(
                    ctx, woT_c, (((1,), (1,)), ((), ())),
                    preferred_element_type=jnp.float32)
                if first:
                    out_ref[b] = contrib
                else:
                    out_ref[b] = out_ref[b] + contrib

        for half in range(2):
            start_send(0, 1, half, send_f.at[0, half], right)
            start_send(0, 5, half, send_b.at[0, half], left)
            start_send(0, 4, half, send_z.at[half], partner)
        for dma in kv_dmas:
            dma.wait()
        compute_chunk(0, my_pos, first=True)

        for r in range(2, FWD + 1):
            for half in range(2):
                wait_recv(r - 1, half)
                start_send(r - 1, r, half, send_f.at[r - 1, half], right)
            bwd_slot = 4 + (r - 1)
            for half in range(2):
                wait_recv(bwd_slot, half)
                if r - 1 < BWD:
                    start_send(bwd_slot, bwd_slot + 1, half,
                               send_b.at[r - 1, half], left)
            compute_chunk(r - 1, my_pos - (r - 1))
            compute_chunk(bwd_slot, my_pos + (r - 1))
            if r == 2:
                for half in range(2):
                    wait_recv(4, half)
                compute_chunk(4, my_pos + 4)

        for half in range(2):
            wait_recv(FWD, half)
            wait_recv(4 + BWD, half)
        compute_chunk(FWD, my_pos - FWD)
        compute_chunk(4 + BWD, my_pos + BWD)
        for r in sends:
            r.wait_send()

    out_shape = jax.ShapeDtypeStruct((B_LOC, SQ, D_MODEL), jnp.float32)
    return pl.pallas_call(
        body,
        out_shape=out_shape,
        in_specs=[
            pl.BlockSpec(memory_space=pltpu.VMEM),
            pl.BlockSpec(memory_space=pltpu.VMEM),
            pl.BlockSpec(memory_space=pltpu.MemorySpace.HBM),
            pl.BlockSpec(memory_space=pltpu.MemorySpace.HBM),
        ],
        out_specs=pl.BlockSpec(memory_space=pltpu.VMEM),
        scratch_shapes=[
            pltpu.VMEM((N_DEV, 2, D_MODEL, D_CHUNK), jnp.bfloat16),
            pltpu.VMEM((2, B_LOC, SKV, HQ * DH), jnp.bfloat16),
            pltpu.SemaphoreType.DMA((FWD, 2)),
            pltpu.SemaphoreType.DMA((N_DEV, 2)),
            pltpu.SemaphoreType.DMA((BWD, 2)),
            pltpu.SemaphoreType.DMA((2,)),
            pltpu.SemaphoreType.DMA((2,)),
        ],
        compiler_params=pltpu.CompilerParams(collective_id=0),
    )(x_b, chunk, k_b, v_b)


# device time: 41486 ns/iter; 1.1438x vs baseline; 1.1438x over previous
import jax
import jax.numpy as jnp
from jax import lax
from jax.experimental import pallas as pl
from jax.experimental.pallas import tpu as pltpu

N_DEV = 8
FWD = 4
BWD = 3
B_LOC = 2
H_LOC = 4
SQ = 128
SKV = 128
HQ = 32
DH = 64
D_MODEL = 512
D_CHUNK = H_LOC * DH


def kernel(x, Wq, K_ext, V_ext, Wo):
    my = lax.axis_index("i")
    k2 = K_ext.reshape(K_ext.shape[0], SKV, HQ * DH)
    v2 = V_ext.reshape(V_ext.shape[0], SKV, HQ * DH)
    k_b = lax.dynamic_slice_in_dim(k2, my * B_LOC, B_LOC, axis=0).astype(
        jnp.bfloat16)
    v_b = lax.dynamic_slice_in_dim(v2, my * B_LOC, B_LOC, axis=0).astype(
        jnp.bfloat16)
    x_b = x.astype(jnp.bfloat16)
    chunk = jnp.stack(
        [Wq.astype(jnp.bfloat16), Wo.T.astype(jnp.bfloat16)])

    def body(x_ref, chunk_ref, k_hbm, v_hbm, out_ref,
             comm, kvbuf, send_f, recv_sems, send_b, kv_sems):
        my_pos = lax.axis_index("i")
        left = jnp.mod(my_pos - 1, N_DEV)
        right = jnp.mod(my_pos + 1, N_DEV)

        kv_dmas = []
        for t, hbm in ((0, k_hbm), (1, v_hbm)):
            dma = pltpu.make_async_copy(hbm, kvbuf.at[t], kv_sems.at[t])
            dma.start()
            kv_dmas.append(dma)

        barrier_sem = pltpu.get_barrier_semaphore()
        for nbr in (left, right):
            pl.semaphore_signal(
                barrier_sem, inc=1,
                device_id=(nbr,), device_id_type=pl.DeviceIdType.MESH,
            )
        pl.semaphore_wait(barrier_sem, 2)

        comm[0] = chunk_ref[...]

        qb = lax.broadcasted_iota(jnp.int32, (SQ, SQ), 0) // 64
        kb = lax.broadcasted_iota(jnp.int32, (SQ, SQ), 1) // 64
        mask = (qb == kb) | ((kb % 4) == (qb % 4))

        sends = []

        def start_send(src_slot, dst_slot, half, sem, target):
            r = pltpu.make_async_remote_copy(
                src_ref=comm.at[src_slot, half],
                dst_ref=comm.at[dst_slot, half],
                send_sem=sem, recv_sem=recv_sems.at[dst_slot, half],
                device_id=(target,), device_id_type=pl.DeviceIdType.MESH,
            )
            r.start()
            sends.append(r)

        def wait_recv(slot, half):
            pltpu.make_async_remote_copy(
                src_ref=comm.at[0, half], dst_ref=comm.at[slot, half],
                send_sem=send_f.at[0, half],
                recv_sem=recv_sems.at[slot, half],
                device_id=(my_pos,), device_id_type=pl.DeviceIdType.MESH,
            ).wait_recv()

        x2 = x_ref[...].reshape(B_LOC * SQ, D_MODEL)

        def compute_chunk(slot, origin, first=False):
            src = jnp.mod(origin, N_DEV)
            lane0 = src * D_CHUNK
            wq_c = comm[slot, 0]
            woT_c = comm[slot, 1]
            q2 = (jnp.dot(x2, wq_c, preferred_element_type=jnp.float32)
                  ).astype(jnp.bfloat16)
            ctx_rows = []
            for b in range(B_LOC):
                k4 = kvbuf[0, b, :, pl.ds(lane0, D_CHUNK)]
                v4 = kvbuf[1, b, :, pl.ds(lane0, D_CHUNK)]
                rows = slice(b * SQ, (b + 1) * SQ)
                ctx_parts = []
                for h in range(H_LOC):
                    sl = slice(h * DH, (h + 1) * DH)
                    sc = lax.dot_general(
                        q2[rows, sl], k4[:, sl], (((1,), (1,)), ((), ())),
                        preferred_element_type=jnp.float32) * 0.125
                    sc = jnp.where(mask, sc, -1e9)
                    m = jnp.max(sc, axis=-1, keepdims=True)
                    w = jnp.exp(sc - m)
                    w = (w / jnp.sum(w, axis=-1, keepdims=True)
                         ).astype(jnp.bfloat16)
                    ctx_parts.append(
                        jnp.dot(w, v4[:, sl],
                                preferred_element_type=jnp.float32))
                ctx_rows.append(
                    jnp.concatenate(ctx_parts, axis=-1))
            ctx2 = jnp.concatenate(ctx_rows, axis=0).astype(
                jnp.bfloat16)
            contrib2 = lax.dot_general(
                ctx2, woT_c, (((1,), (1,)), ((), ())),
                preferred_element_type=jnp.float32)
            for b in range(B_LOC):
                c = contrib2[b * SQ:(b + 1) * SQ, :]
                if first:
                    out_ref[b] = c
                else:
                    out_ref[b] = out_ref[b] + c

        for half in range(2):
            start_send(0, 1, half, send_f.at[0, half], right)
            start_send(0, 5, half, send_b.at[0, half], left)
        for dma in kv_dmas:
            dma.wait()
        compute_chunk(0, my_pos, first=True)

        for r in range(2, FWD + 1):
            for half in range(2):
                wait_recv(r - 1, half)
                if r < FWD or half == 0:
                    start_send(r - 1, r, half, send_f.at[r - 1, half],
                               right)
            bwd_slot = 4 + (r - 1)
            for half in range(2):
                wait_recv(bwd_slot, half)
                if r - 1 < BWD:
                    start_send(bwd_slot, bwd_slot + 1, half,
                               send_b.at[r - 1, half], left)
                elif half == 1:
                    start_send(bwd_slot, 4, half, send_b.at[r - 1, half],
                               left)
            compute_chunk(r - 1, my_pos - (r - 1))
            compute_chunk(bwd_slot, my_pos + (r - 1))

        for half in range(2):
            wait_recv(FWD, half)
        compute_chunk(FWD, my_pos - FWD)
        for r in sends:
            r.wait_send()

    out_shape = jax.ShapeDtypeStruct((B_LOC, SQ, D_MODEL), jnp.float32)
    return pl.pallas_call(
        body,
        out_shape=out_shape,
        in_specs=[
            pl.BlockSpec(memory_space=pltpu.VMEM),
            pl.BlockSpec(memory_space=pltpu.VMEM),
            pl.BlockSpec(memory_space=pltpu.MemorySpace.HBM),
            pl.BlockSpec(memory_space=pltpu.MemorySpace.HBM),
        ],
        out_specs=pl.BlockSpec(memory_space=pltpu.VMEM),
        scratch_shapes=[
            pltpu.VMEM((N_DEV, 2, D_MODEL, D_CHUNK), jnp.bfloat16),
            pltpu.VMEM((2, B_LOC, SKV, HQ * DH), jnp.bfloat16),
            pltpu.SemaphoreType.DMA((FWD, 2)),
            pltpu.SemaphoreType.DMA((N_DEV, 2)),
            pltpu.SemaphoreType.DMA((FWD, 2)),
            pltpu.SemaphoreType.DMA((2,)),
        ],
        compiler_params=pltpu.CompilerParams(collective_id=0),
    )(x_b, chunk, k_b, v_b)


# device time: 38059 ns/iter; 1.2467x vs baseline; 1.0900x over previous
import jax
import jax.numpy as jnp
from jax import lax
from jax.experimental import pallas as pl
from jax.experimental.pallas import tpu as pltpu

N_DEV = 8
FWD = 4
BWD = 3
B_LOC = 2
H_LOC = 4
SQ = 128
SKV = 128
HQ = 32
DH = 64
D_MODEL = 512
D_CHUNK = H_LOC * DH


def kernel(x, Wq, K_ext, V_ext, Wo):
    my = lax.axis_index("i")
    k2 = K_ext.reshape(K_ext.shape[0], SKV, HQ * DH)
    v2 = V_ext.reshape(V_ext.shape[0], SKV, HQ * DH)
    k_b = lax.dynamic_slice_in_dim(k2, my * B_LOC, B_LOC, axis=0).astype(
        jnp.bfloat16)
    v_b = lax.dynamic_slice_in_dim(v2, my * B_LOC, B_LOC, axis=0).astype(
        jnp.bfloat16)
    x_b = x.astype(jnp.bfloat16)
    chunk = jnp.stack(
        [Wq.astype(jnp.bfloat16), Wo.T.astype(jnp.bfloat16)])

    def body(x_ref, chunk_ref, k_hbm, v_hbm, out_ref,
             comm, kvbuf, send_f, recv_sems, send_b, kv_sems):
        my_pos = lax.axis_index("i")
        left = jnp.mod(my_pos - 1, N_DEV)
        right = jnp.mod(my_pos + 1, N_DEV)

        kv_dmas = []
        for t, hbm in ((0, k_hbm), (1, v_hbm)):
            dma = pltpu.make_async_copy(hbm, kvbuf.at[t], kv_sems.at[t])
            dma.start()
            kv_dmas.append(dma)

        barrier_sem = pltpu.get_barrier_semaphore()
        for nbr in (left, right):
            pl.semaphore_signal(
                barrier_sem, inc=1,
                device_id=(nbr,), device_id_type=pl.DeviceIdType.MESH,
            )
        pl.semaphore_wait(barrier_sem, 2)

        comm[0] = chunk_ref[...]

        qb = lax.broadcasted_iota(jnp.int32, (SQ, SQ), 0) // 64
        kb = lax.broadcasted_iota(jnp.int32, (SQ, SQ), 1) // 64
        mask = (qb == kb) | ((kb % 4) == (qb % 4))
        mask_bias = jnp.where(mask, 0.0, -1e9).astype(jnp.float32)

        sends = []

        def start_send(src_slot, dst_slot, half, sem, target):
            r = pltpu.make_async_remote_copy(
                src_ref=comm.at[src_slot, half],
                dst_ref=comm.at[dst_slot, half],
                send_sem=sem, recv_sem=recv_sems.at[dst_slot, half],
                device_id=(target,), device_id_type=pl.DeviceIdType.MESH,
            )
            r.start()
            sends.append(r)

        def wait_recv(slot, half):
            pltpu.make_async_remote_copy(
                src_ref=comm.at[0, half], dst_ref=comm.at[slot, half],
                send_sem=send_f.at[0, half],
                recv_sem=recv_sems.at[slot, half],
                device_id=(my_pos,), device_id_type=pl.DeviceIdType.MESH,
            ).wait_recv()

        x2 = x_ref[...].reshape(B_LOC * SQ, D_MODEL)

        def compute_chunk(slot, origin, first=False):
            src = jnp.mod(origin, N_DEV)
            lane0 = src * D_CHUNK
            wq_c = comm[slot, 0]
            woT_c = comm[slot, 1]
            q2 = (jnp.dot(x2, wq_c, preferred_element_type=jnp.float32)
                  ).astype(jnp.bfloat16)
            ctx_rows = []
            for b in range(B_LOC):
                k4 = kvbuf[0, b, :, pl.ds(lane0, D_CHUNK)]
                v4 = kvbuf[1, b, :, pl.ds(lane0, D_CHUNK)]
                rows = slice(b * SQ, (b + 1) * SQ)
                ctx_parts = []
                for h in range(H_LOC):
                    sl = slice(h * DH, (h + 1) * DH)
                    sc = lax.dot_general(
                        q2[rows, sl], k4[:, sl], (((1,), (1,)), ((), ())),
                        preferred_element_type=jnp.float32)
                    w = jnp.exp(sc * 0.125 + mask_bias)
                    rnorm = 1.0 / jnp.sum(w, axis=-1, keepdims=True)
                    ctx_parts.append(
                        jnp.dot(w.astype(jnp.bfloat16), v4[:, sl],
                                preferred_element_type=jnp.float32)
                        * rnorm)
                ctx_rows.append(
                    jnp.concatenate(ctx_parts, axis=-1))
            ctx2 = jnp.concatenate(ctx_rows, axis=0).astype(
                jnp.bfloat16)
            contrib2 = lax.dot_general(
                ctx2, woT_c, (((1,), (1,)), ((), ())),
                preferred_element_type=jnp.float32)
            for b in range(B_LOC):
                c = contrib2[b * SQ:(b + 1) * SQ, :]
                if first:
                    out_ref[b] = c
                else:
                    out_ref[b] = out_ref[b] + c

        for half in range(2):
            start_send(0, 1, half, send_f.at[0, half], right)
            start_send(0, 5, half, send_b.at[0, half], left)
        for dma in kv_dmas:
            dma.wait()
        compute_chunk(0, my_pos, first=True)

        for r in range(2, FWD + 1):
            for half in range(2):
                wait_recv(r - 1, half)
                if r < FWD or half == 0:
                    start_send(r - 1, r, half, send_f.at[r - 1, half],
                               right)
            bwd_slot = 4 + (r - 1)
            for half in range(2):
                wait_recv(bwd_slot, half)
                if r - 1 < BWD:
                    start_send(bwd_slot, bwd_slot + 1, half,
                               send_b.at[r - 1, half], left)
                elif half == 1:
                    start_send(bwd_slot, 4, half, send_b.at[r - 1, half],
                               left)
            compute_chunk(r - 1, my_pos - (r - 1))
            compute_chunk(bwd_slot, my_pos + (r - 1))

        for half in range(2):
            wait_recv(FWD, half)
        compute_chunk(FWD, my_pos - FWD)
        for r in sends:
            r.wait_send()

    out_shape = jax.ShapeDtypeStruct((B_LOC, SQ, D_MODEL), jnp.float32)
    return pl.pallas_call(
        body,
        out_shape=out_shape,
        in_specs=[
            pl.BlockSpec(memory_space=pltpu.VMEM),
            pl.BlockSpec(memory_space=pltpu.VMEM),
            pl.BlockSpec(memory_space=pltpu.MemorySpace.HBM),
            pl.BlockSpec(memory_space=pltpu.MemorySpace.HBM),
        ],
        out_specs=pl.BlockSpec(memory_space=pltpu.VMEM),
        scratch_shapes=[
            pltpu.VMEM((N_DEV, 2, D_MODEL, D_CHUNK), jnp.bfloat16),
            pltpu.VMEM((2, B_LOC, SKV, HQ * DH), jnp.bfloat16),
            pltpu.SemaphoreType.DMA((FWD, 2)),
            pltpu.SemaphoreType.DMA((N_DEV, 2)),
            pltpu.SemaphoreType.DMA((FWD, 2)),
            pltpu.SemaphoreType.DMA((2,)),
        ],
        compiler_params=pltpu.CompilerParams(collective_id=0),
    )(x_b, chunk, k_b, v_b)


# device time: 35986 ns/iter; 1.3186x vs baseline; 1.0576x over previous
import jax
import jax.numpy as jnp
from jax import lax
from jax.experimental import pallas as pl
from jax.experimental.pallas import tpu as pltpu

N_DEV = 8
FWD = 4
BWD = 3
B_LOC = 2
H_LOC = 4
SQ = 128
SKV = 128
HQ = 32
DH = 64
D_MODEL = 512
D_CHUNK = H_LOC * DH


def kernel(x, Wq, K_ext, V_ext, Wo):
    my = lax.axis_index("i")
    k2 = K_ext.reshape(K_ext.shape[0], SKV, HQ * DH)
    v2 = V_ext.reshape(V_ext.shape[0], SKV, HQ * DH)
    k_b = lax.dynamic_slice_in_dim(k2, my * B_LOC, B_LOC, axis=0).astype(
        jnp.bfloat16)
    v_b = lax.dynamic_slice_in_dim(v2, my * B_LOC, B_LOC, axis=0).astype(
        jnp.bfloat16)
    x_b = x.astype(jnp.bfloat16)
    chunk = jnp.stack(
        [(Wq * 0.125).astype(jnp.bfloat16), Wo.T.astype(jnp.bfloat16)])

    def body(x_ref, chunk_ref, k_hbm, v_hbm, out_ref,
             comm, kvbuf, send_f, recv_sems, send_b, kv_sems):
        my_pos = lax.axis_index("i")
        left = jnp.mod(my_pos - 1, N_DEV)
        right = jnp.mod(my_pos + 1, N_DEV)

        kv_dmas = []
        for t, hbm in ((0, k_hbm), (1, v_hbm)):
            dma = pltpu.make_async_copy(hbm, kvbuf.at[t], kv_sems.at[t])
            dma.start()
            kv_dmas.append(dma)

        barrier_sem = pltpu.get_barrier_semaphore()
        for nbr in (left, right):
            pl.semaphore_signal(
                barrier_sem, inc=1,
                device_id=(nbr,), device_id_type=pl.DeviceIdType.MESH,
            )
        pl.semaphore_wait(barrier_sem, 2)

        comm[0] = chunk_ref[...]

        qb = lax.broadcasted_iota(jnp.int32, (SQ, SQ), 0) // 64
        kb = lax.broadcasted_iota(jnp.int32, (SQ, SQ), 1) // 64
        mask = (qb == kb) | ((kb % 4) == (qb % 4))
        mask_bias = jnp.where(mask, 0.0, -1e9).astype(jnp.float32)

        sends = []

        def start_send(src_slot, dst_slot, half, sem, target):
            r = pltpu.make_async_remote_copy(
                src_ref=comm.at[src_slot, half],
                dst_ref=comm.at[dst_slot, half],
                send_sem=sem, recv_sem=recv_sems.at[dst_slot, half],
                device_id=(target,), device_id_type=pl.DeviceIdType.MESH,
            )
            r.start()
            sends.append(r)

        def wait_recv(slot, half):
            pltpu.make_async_remote_copy(
                src_ref=comm.at[0, half], dst_ref=comm.at[slot, half],
                send_sem=send_f.at[0, half],
                recv_sem=recv_sems.at[slot, half],
                device_id=(my_pos,), device_id_type=pl.DeviceIdType.MESH,
            ).wait_recv()

        x2 = x_ref[...].reshape(B_LOC * SQ, D_MODEL)

        def attn(slot, origin):
            src = jnp.mod(origin, N_DEV)
            lane0 = src * D_CHUNK
            wq_c = comm[slot, 0]
            q2 = (jnp.dot(x2, wq_c, preferred_element_type=jnp.float32)
                  ).astype(jnp.bfloat16)
            ctx_rows = []
            for b in range(B_LOC):
                k4 = kvbuf[0, b, :, pl.ds(lane0, D_CHUNK)]
                v4 = kvbuf[1, b, :, pl.ds(lane0, D_CHUNK)]
                rows = slice(b * SQ, (b + 1) * SQ)
                ctx_parts = []
                for h in range(H_LOC):
                    sl = slice(h * DH, (h + 1) * DH)
                    sc = lax.dot_general(
                        q2[rows, sl], k4[:, sl], (((1,), (1,)), ((), ())),
                        preferred_element_type=jnp.float32)
                    w = jnp.exp(sc + mask_bias)
                    rnorm = 1.0 / jnp.sum(w, axis=-1, keepdims=True)
                    ctx_parts.append(
                        jnp.dot(w.astype(jnp.bfloat16), v4[:, sl],
                                preferred_element_type=jnp.float32)
                        * rnorm)
                ctx_rows.append(
                    jnp.concatenate(ctx_parts, axis=-1))
            return jnp.concatenate(ctx_rows, axis=0).astype(
                jnp.bfloat16)

        def proj(slot, ctx2, first=False):
            woT_c = comm[slot, 1]
            contrib2 = lax.dot_general(
                ctx2, woT_c, (((1,), (1,)), ((), ())),
                preferred_element_type=jnp.float32)
            for b in range(B_LOC):
                c = contrib2[b * SQ:(b + 1) * SQ, :]
                if first:
                    out_ref[b] = c
                else:
                    out_ref[b] = out_ref[b] + c

        for half in range(2):
            start_send(0, 1, half, send_f.at[0, half], right)
            start_send(0, 5, half, send_b.at[0, half], left)
        for dma in kv_dmas:
            dma.wait()
        proj(0, attn(0, my_pos), first=True)

        for r in range(2, FWD + 1):
            fwd_slot = r - 1
            bwd_slot = 4 + (r - 1)
            wait_recv(fwd_slot, 0)
            start_send(fwd_slot, r, 0, send_f.at[r - 1, 0], right)
            wait_recv(bwd_slot, 0)
            if r - 1 < BWD:
                start_send(bwd_slot, bwd_slot + 1, 0,
                           send_b.at[r - 1, 0], left)
            ctx_f = attn(fwd_slot, my_pos - (r - 1))
            ctx_b = attn(bwd_slot, my_pos + (r - 1))
            wait_recv(fwd_slot, 1)
            if r < FWD:
                start_send(fwd_slot, r, 1, send_f.at[r - 1, 1], right)
            wait_recv(bwd_slot, 1)
            if r - 1 < BWD:
                start_send(bwd_slot, bwd_slot + 1, 1,
                           send_b.at[r - 1, 1], left)
            else:
                start_send(bwd_slot, 4, 1, send_b.at[r - 1, 1], left)
            proj(fwd_slot, ctx_f)
            proj(bwd_slot, ctx_b)

        wait_recv(FWD, 0)
        ctx_a = attn(FWD, my_pos - FWD)
        wait_recv(FWD, 1)
        proj(FWD, ctx_a)
        for r in sends:
            r.wait_send()

    out_shape = jax.ShapeDtypeStruct((B_LOC, SQ, D_MODEL), jnp.float32)
    return pl.pallas_call(
        body,
        out_shape=out_shape,
        in_specs=[
            pl.BlockSpec(memory_space=pltpu.VMEM),
            pl.BlockSpec(memory_space=pltpu.VMEM),
            pl.BlockSpec(memory_space=pltpu.MemorySpace.HBM),
            pl.BlockSpec(memory_space=pltpu.MemorySpace.HBM),
        ],
        out_specs=pl.BlockSpec(memory_space=pltpu.VMEM),
        scratch_shapes=[
            pltpu.VMEM((N_DEV, 2, D_MODEL, D_CHUNK), jnp.bfloat16),
            pltpu.VMEM((2, B_LOC, SKV, HQ * DH), jnp.bfloat16),
            pltpu.SemaphoreType.DMA((FWD, 2)),
            pltpu.SemaphoreType.DMA((N_DEV, 2)),
            pltpu.SemaphoreType.DMA((FWD, 2)),
            pltpu.SemaphoreType.DMA((2,)),
        ],
        compiler_params=pltpu.CompilerParams(collective_id=0),
    )(x_b, chunk, k_b, v_b)
